# Initial kernel scaffold; baseline (speedup 1.0000x reference)
import jax
import jax.numpy as jnp
from jax.experimental import pallas as pl


def kernel(inputs, targets):
    raise NotImplementedError("write your pallas kernel here")


# speedup vs baseline: 1.1026x; 1.1026x over previous
import jax
import jax.numpy as jnp
from jax.experimental import pallas as pl

MARGIN_ = 0.3
N_ = 4096
D_ = 64
BLK_ = 512


def _loss_body(a_blk_ref, a_all_ref, t_row_ref, t_col_ref, out_ref):
    i = pl.program_id(0)
    nblk = pl.num_programs(0)

    a_blk = a_blk_ref[...]
    a_all = a_all_ref[...]
    sim = jax.lax.dot_general(
        a_blk, a_all,
        dimension_numbers=(((1,), (1,)), ((), ())),
        preferred_element_type=jnp.float32,
    )

    same = t_row_ref[...] == t_col_ref[...]
    contrib = jnp.where(
        same,
        jnp.where(sim < 1.0, 1.0 - sim, 0.0),
        jnp.where(sim > MARGIN_, sim, 0.0),
    )
    part = jnp.sum(contrib)[None, None]

    @pl.when(i == 0)
    def _init():
        out_ref[...] = jnp.zeros_like(out_ref)

    out_ref[...] += part

    @pl.when(i == nblk - 1)
    def _finish():
        out_ref[...] = out_ref[...] * (1.0 / N_)


def kernel(inputs, targets):
    n, d = inputs.shape
    t_row = targets.reshape(n, 1)
    t_col = targets.reshape(1, n)
    nblk = n // BLK_

    out = pl.pallas_call(
        _loss_body,
        grid=(nblk,),
        in_specs=[
            pl.BlockSpec((BLK_, d), lambda i: (i, 0)),
            pl.BlockSpec((n, d), lambda i: (0, 0)),
            pl.BlockSpec((BLK_, 1), lambda i: (i, 0)),
            pl.BlockSpec((1, n), lambda i: (0, 0)),
        ],
        out_specs=pl.BlockSpec((1, 1), lambda i: (0, 0)),
        out_shape=jax.ShapeDtypeStruct((1, 1), jnp.float32),
    )(inputs, inputs, t_row, t_col)
    return out[0, 0]
